# baseline (speedup 1.0000x reference)
import functools

import jax
import jax.numpy as jnp
from jax import lax
from jax.experimental import pallas as pl
from jax.experimental.pallas import tpu as pltpu
from jax.experimental.pallas import tpu_sc as plsc

NUM_EMB = 1_000_000
DIM = 32
B_TOTAL = 4096 * 200

_info = plsc.get_sparse_core_info()
NC = _info.num_cores
NS = _info.num_subcores
NW = NC * NS
B_PER_W = B_TOTAL // NW
CHUNK = 1600
NCHUNK = B_PER_W // CHUNK


def _emb_body(idx_hbm, tbl_hbm, out_hbm,
              idx_v, rows_v0, rows_v1,
              sg0, sg1, so0, so1):
    rows_v = (rows_v0, rows_v1)
    sg = (sg0, sg1)
    so = (so0, so1)

    wid = lax.axis_index("s") * NC + lax.axis_index("c")
    base = wid * B_PER_W

    def idx_slice(c):
        return idx_v.at[pl.ds(c * CHUNK, CHUNK)]

    def out_slice(c):
        return out_hbm.at[pl.ds(base + c * CHUNK, CHUNK)]

    pltpu.sync_copy(idx_hbm.at[pl.ds(base, B_PER_W)], idx_v)

    pltpu.async_copy(tbl_hbm.at[idx_slice(0)], rows_v[0], sg[0])

    for c in range(1, NCHUNK):
        b = c % 2
        if c >= 2:
            pltpu.make_async_copy(rows_v[b], out_slice(c - 2), so[b]).wait()
        pltpu.async_copy(tbl_hbm.at[idx_slice(c)], rows_v[b], sg[b])
        b1 = 1 - b
        pltpu.make_async_copy(tbl_hbm.at[idx_slice(c - 1)], rows_v[b1], sg[b1]).wait()
        pltpu.async_copy(rows_v[b1], out_slice(c - 1), so[b1])

    bl = (NCHUNK - 1) % 2
    pltpu.make_async_copy(tbl_hbm.at[idx_slice(NCHUNK - 1)], rows_v[bl], sg[bl]).wait()
    pltpu.async_copy(rows_v[bl], out_slice(NCHUNK - 1), so[bl])
    pltpu.make_async_copy(rows_v[1 - bl], out_slice(NCHUNK - 2), so[1 - bl]).wait()
    pltpu.make_async_copy(rows_v[bl], out_slice(NCHUNK - 1), so[bl]).wait()


_emb_call = functools.partial(
    pl.kernel,
    mesh=plsc.VectorSubcoreMesh(core_axis_name="c", subcore_axis_name="s"),
    out_type=jax.ShapeDtypeStruct((B_TOTAL, DIM), jnp.float32),
    scratch_types=[
        pltpu.VMEM((B_PER_W,), jnp.int32),
        pltpu.VMEM((CHUNK, DIM), jnp.float32),
        pltpu.VMEM((CHUNK, DIM), jnp.float32),
        pltpu.SemaphoreType.DMA,
        pltpu.SemaphoreType.DMA,
        pltpu.SemaphoreType.DMA,
        pltpu.SemaphoreType.DMA,
    ],
    compiler_params=pltpu.CompilerParams(use_tc_tiling_on_sc=False),
)(_emb_body)


@jax.jit
def kernel(token_ids, weight):
    idx = token_ids.reshape(-1).astype(jnp.int32)
    out = _emb_call(idx, weight)
    return out.reshape(token_ids.shape + (DIM,))
